# baseline (speedup 1.0000x reference)
import functools

import jax
import jax.numpy as jnp
import numpy as np
from jax import lax
from jax.experimental import pallas as pl
from jax.experimental.pallas import tpu as pltpu
from jax.experimental.pallas import tpu_sc as plsc

N = 65536
G = 2048
T = 20
S = 256
SCALE = 1000000.0

BN = 2048
NBLK = N // BN
C = 128
R = 32
SUMCOL = T

NC = 2
NS = 16
CPT = N // (NC * NS)


def _helmert_basis(D):
    H = np.zeros((D - 1, D), dtype=np.float32)
    for i in range(D - 1):
        H[i, : i + 1] = 1.0 / (i + 1)
        H[i, i + 1] = -1.0
        H[i] *= np.sqrt((i + 1) / (i + 2))
    return H


def _dot(a, b, dims):
    return lax.dot_general(a, b, (dims, ((), ())),
                           preferred_element_type=jnp.float32)


def _dense_body(x_ref, w_ref, v_ref, b_ref, y_ref):
    x = x_ref[...]
    lib = jnp.maximum(jnp.sum(x, axis=1, keepdims=True), 1e-8)
    xn = x * (SCALE / lib)
    t = _dot(xn, w_ref[...], (((1,), (0,)))) + b_ref[...]
    logxT = _dot(v_ref[...], t, (((0,), (1,))))
    row = lax.broadcasted_iota(jnp.int32, (R, BN), 0)
    logxT = jnp.where(row < T + 1, logxT, -jnp.inf)
    m = jnp.max(logxT, axis=0, keepdims=True)
    e = jnp.exp(logxT - m)
    y = e / jnp.sum(e, axis=0, keepdims=True)
    y_ref[...] = y.T


def _dense(X_batch, Wp, Vp, bp):
    return pl.pallas_call(
        _dense_body,
        grid=(NBLK,),
        in_specs=[
            pl.BlockSpec((BN, G), lambda i: (i, 0)),
            pl.BlockSpec((G, C), lambda i: (0, 0)),
            pl.BlockSpec((C, R), lambda i: (0, 0)),
            pl.BlockSpec((1, C), lambda i: (0, 0)),
        ],
        out_specs=pl.BlockSpec((BN, R), lambda i: (i, 0)),
        out_shape=jax.ShapeDtypeStruct((N, R), jnp.float32),
        compiler_params=pltpu.CompilerParams(
            dimension_semantics=("arbitrary",),
        ),
    )(X_batch, Wp, Vp, bp)


def _seg_body(y_hbm, idx_hbm, out_hbm, yslab, idxv, acc, rowids, shacc):
    c = lax.axis_index("c")
    s = lax.axis_index("s")
    iota16 = lax.iota(jnp.int32, 16)
    zeros16 = jnp.zeros((16,), jnp.float32)

    def _zero(r, _):
        acc[r, pl.ds(0, 16)] = zeros16
        acc[r, pl.ds(16, 16)] = zeros16
        return 0
    lax.fori_loop(0, S, _zero, 0)

    @pl.when(s == 0)
    def _():
        pltpu.sync_copy(acc, shacc)

    def _ids(g, _):
        rowids[pl.ds(g * 16, 16)] = iota16 + g * 16
        return 0
    lax.fori_loop(0, S // 16, _ids, 0)

    wid = s * NC + c
    base = wid * CPT
    pltpu.sync_copy(y_hbm.at[pl.ds(base, CPT)], yslab)
    pltpu.sync_copy(idx_hbm.at[pl.ds(base, CPT)], idxv)

    def _group(g, _):
        segv = idxv[pl.ds(g * 16, 16)]
        for j in range(16):
            n = g * 16 + j
            seg = segv[j]
            v0 = yslab[n, pl.ds(0, 16)]
            v1 = yslab[n, pl.ds(16, 16)]
            acc[seg, pl.ds(0, 16)] += v0
            acc[seg, pl.ds(16, 16)] += v1
        return 0
    lax.fori_loop(0, CPT // 16, _group, 0)

    plsc.subcore_barrier()
    pltpu.sync_copy(acc, shacc.at[rowids], add=True)
    plsc.subcore_barrier()

    @pl.when(s == 0)
    def _():
        pltpu.sync_copy(shacc, acc)
        pltpu.sync_copy(acc, out_hbm.at[c])


def _segsum(y, batch_idx):
    mesh = plsc.VectorSubcoreMesh(core_axis_name="c", subcore_axis_name="s")
    return pl.kernel(
        _seg_body,
        mesh=mesh,
        compiler_params=pltpu.CompilerParams(use_tc_tiling_on_sc=False),
        out_type=jax.ShapeDtypeStruct((NC, S, R), jnp.float32),
        scratch_types=[
            pltpu.VMEM((CPT, R), jnp.float32),
            pltpu.VMEM((CPT,), jnp.int32),
            pltpu.VMEM((S, R), jnp.float32),
            pltpu.VMEM((S,), jnp.int32),
            pltpu.VMEM_SHARED((S, R), jnp.float32),
        ],
    )(y, batch_idx)


def _combine_body(p_ref, o_ref):
    a = p_ref[0] + p_ref[1]
    denom = jnp.maximum(jnp.sum(a, axis=1, keepdims=True), 1e-8)
    o_ref[...] = a / denom


def _combine(partials):
    return pl.pallas_call(
        _combine_body,
        in_specs=[pl.BlockSpec((NC, S, R), lambda: (0, 0, 0))],
        out_specs=pl.BlockSpec((S, R), lambda: (0, 0)),
        out_shape=jax.ShapeDtypeStruct((S, R), jnp.float32),
    )(partials)


@jax.jit
def kernel(X_batch, batch_idx, W, b):
    Wp = jnp.pad(W, ((0, 0), (0, C - T)))
    V = jnp.asarray(_helmert_basis(T + 1))
    Vp = jnp.pad(V, ((0, C - T), (0, R - (T + 1))))
    bp = jnp.pad(b, (0, C - T)).reshape(1, C)

    y = _dense(X_batch, Wp, Vp, bp)
    partials = _segsum(y, batch_idx.astype(jnp.int32))
    out = _combine(partials)
    return out[:, : T + 1]
